# baseline (speedup 1.0000x reference)
import functools

import jax
import jax.numpy as jnp
from jax import lax
from jax.experimental import pallas as pl
from jax.experimental.pallas import tpu as pltpu
from jax.experimental.pallas import tpu_sc as plsc

_B = 16384
_NW = 32
_BPW = _B // _NW
_CH = 128
_NCH = _BPW // _CH
_ROWS = _B // _CH
_L = 16


def _make_sc_kernel():
    mesh = plsc.VectorSubcoreMesh(core_axis_name="c", subcore_axis_name="s")

    @functools.partial(
        pl.kernel,
        mesh=mesh,
        out_type=jax.ShapeDtypeStruct((_ROWS, _CH), jnp.float32),
        scratch_types=[
            pltpu.VMEM((2 * _NCH, _CH), jnp.int32),
            pltpu.VMEM((_NCH, _CH), jnp.float32),
            pltpu.VMEM((_NCH, _CH), jnp.float32),
            pltpu.VMEM((_L,), jnp.int32),
            pltpu.VMEM((_L,), jnp.float32),
            pltpu.VMEM((_L,), jnp.float32),
            pltpu.VMEM((_NCH, _CH), jnp.float32),
            pltpu.SemaphoreType.DMA,
            pltpu.SemaphoreType.DMA,
        ],
    )
    def body(x_hbm, ut_hbm, mt_hbm, w_hbm, b_hbm, out_hbm,
             idx_v, uv_v, mv_v, z_v, w_v, b_v, o_v, sem, sem2):
        wid = lax.axis_index("s") * 2 + lax.axis_index("c")
        r0 = wid * _NCH
        z_v[...] = lax.iota(jnp.int32, _L) * 0
        pltpu.sync_copy(x_hbm.at[pl.ds(2 * r0, 2 * _NCH)], idx_v)
        hw = pltpu.async_copy(w_hbm.at[z_v], w_v, sem2)
        hb = pltpu.async_copy(b_hbm.at[z_v], b_v, sem2)
        ut = ut_hbm.at[0]
        mt = mt_hbm.at[0]
        handles = []
        for j in range(_NCH):
            uid = idx_v.at[2 * j]
            mid = idx_v.at[2 * j + 1]
            handles.append(pltpu.async_copy(ut.at[uid], uv_v.at[j], sem))
            handles.append(pltpu.async_copy(mt.at[mid], mv_v.at[j], sem))
        hw.wait()
        hb.wait()
        w = w_v[...]
        b = b_v[...]
        for j in range(_NCH):
            handles[2 * j].wait()
            handles[2 * j + 1].wait()
            for i in range(_CH // _L):
                sl = pl.ds(i * _L, _L)
                z = uv_v[j, sl] * mv_v[j, sl] * w + b
                o_v[j, sl] = 1.0 / (1.0 + jnp.exp(-z))
        pltpu.sync_copy(o_v, out_hbm.at[pl.ds(r0, _NCH)])

    return body


_SC_KERNEL = _make_sc_kernel()


def kernel(x, u_table, m_table, fc_w, fc_b):
    xb = x.reshape(_ROWS, _CH, 2).transpose(0, 2, 1).reshape(2 * _ROWS, _CH)
    ut = u_table.reshape(1, -1)
    mt = m_table.reshape(1, -1)
    out = _SC_KERNEL(xb, ut, mt, fc_w.reshape(1), fc_b)
    return out.reshape(_B, 1)


# speedup vs baseline: 1.0312x; 1.0312x over previous
import functools

import jax
import jax.numpy as jnp
from jax import lax
from jax.experimental import pallas as pl
from jax.experimental.pallas import tpu as pltpu
from jax.experimental.pallas import tpu_sc as plsc

_B = 16384
_NW = 32
_BPW = _B // _NW
_CH = 128
_NCH = _BPW // _CH
_ROWS = _B // _CH
_L = 16


def _make_sc_kernel():
    mesh = plsc.VectorSubcoreMesh(core_axis_name="c", subcore_axis_name="s")

    @functools.partial(
        pl.kernel,
        mesh=mesh,
        out_type=jax.ShapeDtypeStruct((_ROWS, _CH), jnp.float32),
        scratch_types=[
            pltpu.VMEM((2 * _NCH, _CH), jnp.int32),
            pltpu.VMEM((_NCH, _CH), jnp.float32),
            pltpu.VMEM((_NCH, _CH), jnp.float32),
            pltpu.SemaphoreType.DMA,
        ],
    )
    def body(x_hbm, ut_hbm, mt_hbm, out_hbm, idx_v, uv_v, mv_v, sem):
        wid = lax.axis_index("s") * 2 + lax.axis_index("c")
        r0 = wid * _NCH
        pltpu.sync_copy(x_hbm.at[pl.ds(2 * r0, 2 * _NCH)], idx_v)
        ut = ut_hbm.at[0]
        mt = mt_hbm.at[0]
        handles = []
        for j in range(_NCH):
            uid = idx_v.at[2 * j]
            mid = idx_v.at[2 * j + 1]
            handles.append(pltpu.async_copy(ut.at[uid], uv_v.at[j], sem))
            handles.append(pltpu.async_copy(mt.at[mid], mv_v.at[j], sem))
        for j in range(_NCH):
            handles[2 * j].wait()
            handles[2 * j + 1].wait()
            for i in range(_CH // _L):
                sl = pl.ds(i * _L, _L)
                uv_v[j, sl] = uv_v[j, sl] * mv_v[j, sl]
        pltpu.sync_copy(uv_v, out_hbm.at[pl.ds(r0, _NCH)])

    return body


_SC_KERNEL = _make_sc_kernel()


def kernel(x, u_table, m_table, fc_w, fc_b):
    xb = x.reshape(_ROWS, _CH, 2).transpose(0, 2, 1).reshape(2 * _ROWS, _CH)
    ut = u_table.reshape(1, -1)
    mt = m_table.reshape(1, -1)
    p = _SC_KERNEL(xb, ut, mt)
    out = jax.nn.sigmoid(p * fc_w.reshape(()) + fc_b[0])
    return out.reshape(_B, 1)


# speedup vs baseline: 1.0356x; 1.0043x over previous
import functools

import jax
import jax.numpy as jnp
from jax import lax
from jax.experimental import pallas as pl
from jax.experimental.pallas import tpu as pltpu
from jax.experimental.pallas import tpu_sc as plsc

_B = 16384
_NW = 32
_BPW = _B // _NW
_CH = 128
_NCH = _BPW // _CH
_ROWS = _B // _CH
_L = 16


def _make_sc_kernel():
    mesh = plsc.VectorSubcoreMesh(core_axis_name="c", subcore_axis_name="s")

    @functools.partial(
        pl.kernel,
        mesh=mesh,
        out_type=jax.ShapeDtypeStruct((_B,), jnp.float32),
        scratch_types=[
            pltpu.VMEM((_BPW,), jnp.int32),
            pltpu.VMEM((_BPW,), jnp.int32),
            pltpu.VMEM((_BPW,), jnp.float32),
            pltpu.VMEM((_BPW,), jnp.float32),
            pltpu.SemaphoreType.DMA,
            pltpu.SemaphoreType.DMA,
        ],
    )
    def body(x_hbm, ut_hbm, mt_hbm, out_hbm, uid_v, mid_v, uv_v, mv_v, semi, sem):
        wid = lax.axis_index("s") * 2 + lax.axis_index("c")
        r0 = 2 * wid * _NCH
        ih = []
        for j in range(_NCH):
            ih.append(pltpu.async_copy(
                x_hbm.at[r0 + 2 * j], uid_v.at[pl.ds(j * _CH, _CH)], semi))
            ih.append(pltpu.async_copy(
                x_hbm.at[r0 + 2 * j + 1], mid_v.at[pl.ds(j * _CH, _CH)], semi))
        for h in ih:
            h.wait()
        hu = pltpu.async_copy(ut_hbm.at[0].at[uid_v], uv_v, sem)
        hm = pltpu.async_copy(mt_hbm.at[0].at[mid_v], mv_v, sem)
        hu.wait()
        hm.wait()
        for i in range(_BPW // _L):
            sl = pl.ds(i * _L, _L)
            uv_v[sl] = uv_v[sl] * mv_v[sl]
        pltpu.sync_copy(uv_v, out_hbm.at[pl.ds(wid * _BPW, _BPW)])

    return body


_SC_KERNEL = _make_sc_kernel()


def kernel(x, u_table, m_table, fc_w, fc_b):
    xb = x.reshape(_ROWS, _CH, 2).transpose(0, 2, 1).reshape(2 * _ROWS, _CH)
    ut = u_table.reshape(1, -1)
    mt = m_table.reshape(1, -1)
    p = _SC_KERNEL(xb, ut, mt)
    out = jax.nn.sigmoid(p * fc_w.reshape(()) + fc_b[0])
    return out.reshape(_B, 1)
